# Initial kernel scaffold; baseline (speedup 1.0000x reference)
import jax
import jax.numpy as jnp
from jax.experimental import pallas as pl


def kernel(x, edge_index, W1, b1, W2, b2, W3, b3):
    raise NotImplementedError("write your pallas kernel here")


# speedup vs baseline: 6.7851x; 6.7851x over previous
import functools

import jax
import jax.numpy as jnp
from jax import lax
from jax.experimental import pallas as pl
from jax.experimental.pallas import tpu as pltpu
from jax.experimental.pallas import tpu_sc as plsc

N = 10000
E = 320000
D = 128
NC, NS = 2, 16
C = 128
GA = 158
EPA = NS * GA * C
NPC = 5120
ACC_R = NPC + C
ZPT = ACC_R // NS
DPT = NPC // NS
OROWS = NC * NPC
R = 1000
GRID = N // R


def _fill_f32(ref, val):
    rows, cols = ref.shape

    @pl.loop(0, rows)
    def _(r):
        for q in range(cols // 16):
            ref[r, pl.ds(q * 16, 16)] = jnp.full((16,), val, jnp.float32)


_SC_MESH = plsc.VectorSubcoreMesh(
    core_axis_name="c", subcore_axis_name="s", num_cores=NC, num_subcores=NS
)


def _zero_acc(zeros_buf, acc, s):
    _fill_f32(zeros_buf, 0.0)
    zb = s * ZPT
    pltpu.sync_copy(zeros_buf, acc.at[pl.ds(zb, C)])
    pltpu.sync_copy(zeros_buf, acc.at[pl.ds(zb + C, C)])
    pltpu.sync_copy(zeros_buf.at[pl.ds(0, ZPT - 2 * C)],
                    acc.at[pl.ds(zb + 2 * C, ZPT - 2 * C)])


def _remap_dst(dst_v, c):
    off = jnp.full((16,), NPC, jnp.int32) * c
    lim = jnp.full((16,), NPC, jnp.int32)

    @pl.loop(0, GA)
    def _(g):
        for q in range(C // 16):
            sl = pl.ds(q * 16, 16)
            v = dst_v[g, sl] - off
            ok = (v >= 0) & (v < lim)
            dst_v[g, sl] = jnp.where(ok, v, lim)


def _drain_acc(acc, out_hbm, c, s):
    db = s * DPT
    for lo, n in ((0, C), (C, C), (2 * C, DPT - 2 * C)):
        pltpu.sync_copy(acc.at[pl.ds(db + lo, n)],
                        out_hbm.at[pl.ds(c * NPC + db + lo, n)])


@functools.partial(
    pl.kernel,
    out_type=jax.ShapeDtypeStruct((OROWS, D), jnp.float32),
    mesh=_SC_MESH,
    scratch_types=[
        pltpu.VMEM((GA, C), jnp.int32),
        pltpu.VMEM((C, D), jnp.float32),
        pltpu.VMEM_SHARED((ACC_R, D), jnp.float32),
    ],
)
def _deg_kernel(dsts_hbm, out_hbm, dst_v, ones_v, acc):
    c = lax.axis_index("c")
    s = lax.axis_index("s")
    _zero_acc(ones_v, acc, s)
    _fill_f32(ones_v, 1.0)
    pltpu.sync_copy(dsts_hbm.at[s], dst_v)
    _remap_dst(dst_v, c)
    plsc.subcore_barrier()

    @pl.loop(0, GA)
    def _(g):
        pltpu.sync_copy(ones_v, acc.at[dst_v.at[g]], add=True)

    plsc.subcore_barrier()
    _drain_acc(acc, out_hbm, c, s)


@functools.partial(
    pl.kernel,
    out_type=jax.ShapeDtypeStruct((OROWS, D), jnp.float32),
    mesh=_SC_MESH,
    scratch_types=[
        pltpu.VMEM((GA, C), jnp.int32),
        pltpu.VMEM((GA, C), jnp.int32),
        pltpu.VMEM((C, D), jnp.float32),
        pltpu.VMEM((C, D), jnp.float32),
        pltpu.VMEM_SHARED((ACC_R, D), jnp.float32),
        pltpu.SemaphoreType.DMA,
        pltpu.SemaphoreType.DMA,
    ],
)
def _agg_kernel(y_hbm, srcs_hbm, dsts_hbm, out_hbm, src_v, dst_v, rows0, rows1,
                acc, sem0, sem1):
    c = lax.axis_index("c")
    s = lax.axis_index("s")
    _zero_acc(rows0, acc, s)
    pltpu.sync_copy(srcs_hbm.at[s], src_v)
    pltpu.sync_copy(dsts_hbm.at[s], dst_v)
    _remap_dst(dst_v, c)
    plsc.subcore_barrier()

    pltpu.async_copy(y_hbm.at[src_v.at[0]], rows0, sem0)

    @pl.loop(0, GA, step=2)
    def _(g):
        pltpu.make_async_copy(y_hbm.at[src_v.at[g]], rows0, sem0).wait()
        pltpu.async_copy(y_hbm.at[src_v.at[g + 1]], rows1, sem1)
        pltpu.sync_copy(rows0, acc.at[dst_v.at[g]], add=True)
        pltpu.make_async_copy(y_hbm.at[src_v.at[g + 1]], rows1, sem1).wait()

        @pl.when(g + 2 < GA)
        def _():
            pltpu.async_copy(y_hbm.at[src_v.at[g + 2]], rows0, sem0)

        pltpu.sync_copy(rows1, acc.at[dst_v.at[g + 1]], add=True)

    plsc.subcore_barrier()
    _drain_acc(acc, out_hbm, c, s)


def _dinv_of(deg_blk):
    return lax.rsqrt(1.0 + deg_blk[:, 0:1])


def _first_body(x_ref, w_ref, deg_ref, y_ref):
    dinv = _dinv_of(deg_ref[...])
    y_ref[...] = dinv * jnp.dot(x_ref[...], w_ref[...],
                                preferred_element_type=jnp.float32)


def _mid_body(agg_ref, yprev_ref, deg_ref, b_ref, w_ref, yout_ref):
    dinv = _dinv_of(deg_ref[...])
    agg = agg_ref[...] + yprev_ref[...]
    h = jnp.maximum(dinv * agg + b_ref[...], 0.0)
    yout_ref[...] = dinv * jnp.dot(h, w_ref[...],
                                   preferred_element_type=jnp.float32)


def _final_body(agg_ref, yprev_ref, deg_ref, b_ref, out_ref):
    dinv = _dinv_of(deg_ref[...])
    out_ref[...] = dinv * (agg_ref[...] + yprev_ref[...]) + b_ref[...]


_X_SPEC = pl.BlockSpec((R, D), lambda i: (i, 0))
_W_SPEC = pl.BlockSpec((D, D), lambda i: (0, 0))
_B_SPEC = pl.BlockSpec((1, D), lambda i: (0, 0))
_Y_OUT = jax.ShapeDtypeStruct((N, D), jnp.float32)

_first_call = pl.pallas_call(
    _first_body, grid=(GRID,),
    in_specs=[_X_SPEC, _W_SPEC, _X_SPEC], out_specs=_X_SPEC,
    out_shape=_Y_OUT)

_mid_call = pl.pallas_call(
    _mid_body, grid=(GRID,),
    in_specs=[_X_SPEC, _X_SPEC, _X_SPEC, _B_SPEC, _W_SPEC],
    out_specs=_X_SPEC, out_shape=_Y_OUT)

_final_call = pl.pallas_call(
    _final_body, grid=(GRID,),
    in_specs=[_X_SPEC, _X_SPEC, _X_SPEC, _B_SPEC],
    out_specs=_X_SPEC, out_shape=_Y_OUT)


@jax.jit
def kernel(x, edge_index, W1, b1, W2, b2, W3, b3):
    src = edge_index[0].astype(jnp.int32)
    dst = edge_index[1].astype(jnp.int32)
    srcs = jnp.concatenate(
        [src, jnp.zeros((EPA - E,), jnp.int32)]).reshape(NS, GA, C)
    dsts = jnp.concatenate(
        [dst, jnp.full((EPA - E,), 2 * NPC, jnp.int32)]).reshape(NS, GA, C)
    b1r, b2r, b3r = b1.reshape(1, D), b2.reshape(1, D), b3.reshape(1, D)

    deg = _deg_kernel(dsts)[:N]
    y1 = _first_call(x, W1, deg)
    agg1 = _agg_kernel(y1, srcs, dsts)[:N]
    y2 = _mid_call(agg1, y1, deg, b1r, W2)
    agg2 = _agg_kernel(y2, srcs, dsts)[:N]
    y3 = _mid_call(agg2, y2, deg, b2r, W3)
    agg3 = _agg_kernel(y3, srcs, dsts)[:N]
    return _final_call(agg3, y3, deg, b3r)
